# baseline (speedup 1.0000x reference)
import functools

import jax
import jax.numpy as jnp
from jax import lax
from jax.experimental import pallas as pl
from jax.experimental.pallas import tpu as pltpu
from jax.experimental.pallas import tpu_sc as plsc

N = 10000
E = 320000
D = 128
DH = D // 2
NC = 2
NS = 16
N2 = 10240
STRIPE = N2 // NS
EP = E // NS
KC = 80
Q = 5
SK = KC * Q
NSK = EP // SK
GS = SK // 16


def _pre_body(nf_ref, we_ref, wpn_ref, bpn_ref, be_ref,
              hv_ref, td_ref, ts_ref, lm_ref):
    nf = nf_ref[...]
    hv = (jnp.dot(nf, wpn_ref[...], preferred_element_type=jnp.float32)
          + bpn_ref[...])
    hv_ref[0] = hv[:, :DH]
    hv_ref[1] = hv[:, DH:]
    td = jnp.dot(nf, we_ref[:D, :], preferred_element_type=jnp.float32) + be_ref[0, 0]
    ts = jnp.dot(nf, we_ref[D:, :], preferred_element_type=jnp.float32)
    td_ref[...] = td
    ts_ref[...] = ts
    ub = jnp.max(td) + jnp.max(ts)
    lm = jnp.where(ub >= 0.0, ub, 0.01 * ub)
    lm_ref[...] = jnp.full((8, 128), lm, jnp.float32)


_MESH = plsc.VectorSubcoreMesh(core_axis_name="c", subcore_axis_name="s",
                               num_cores=NC, num_subcores=NS)


@functools.partial(
    pl.kernel,
    out_type=jax.ShapeDtypeStruct((NC, N, DH), jnp.float32),
    mesh=_MESH,
    compiler_params=pltpu.CompilerParams(needs_layout_passes=False,
                                         use_tc_tiling_on_sc=False),
    scratch_types=[
        pltpu.VMEM((N,), jnp.float32),
        pltpu.VMEM((N,), jnp.float32),
        pltpu.VMEM((16,), jnp.float32),
        pltpu.VMEM((SK,), jnp.int32),
        pltpu.VMEM((SK,), jnp.int32),
        pltpu.VMEM((N,), jnp.float32),
        pltpu.VMEM((STRIPE,), jnp.float32),
        pltpu.VMEM((2, SK), jnp.float32),
        pltpu.VMEM((2, Q, KC), jnp.int32),
        pltpu.VMEM((2, Q, KC), jnp.int32),
        pltpu.VMEM((2, Q, KC), jnp.int32),
        pltpu.VMEM((2, SK, DH), jnp.float32),
        pltpu.VMEM_SHARED((N2,), jnp.float32),
        pltpu.VMEM_SHARED((N2, DH), jnp.float32),
        pltpu.SemaphoreType.DMA,
        pltpu.SemaphoreType.DMA,
    ],
)
def _sc_main(td_hbm, ts_hbm, lm_hbm, hv_hbm, src_hbm, dst_hbm, out_hbm,
             td_v, ts_v, lm_v, srcc_v, dstc_v, den_v, zro_v,
             a_v, sidx2, didx2, sdidx2, rows_v, den_sh, c_sh, sem_g, sem_s):
    c = lax.axis_index("c")
    s = lax.axis_index("s")
    zf = jnp.zeros((16,), jnp.float32)

    pltpu.sync_copy(td_hbm, td_v)
    pltpu.sync_copy(ts_hbm, ts_v)
    pltpu.sync_copy(lm_hbm, lm_v)
    e1 = s * EP

    def _z_rows(i, carry):
        for r in range(DH // 16):
            rows_v[0, i, pl.ds(r * 16, 16)] = zf
        return carry
    lax.fori_loop(0, SK, _z_rows, 0)

    def _z(i, carry):
        zro_v[pl.ds(i * 16, 16)] = zf
        return carry
    lax.fori_loop(0, STRIPE // 16, _z, 0)

    row0 = s * STRIPE
    pltpu.sync_copy(zro_v, den_sh.at[pl.ds(row0, STRIPE)])
    pltpu.sync_copy(rows_v.at[0], c_sh.at[pl.ds(row0, SK)])
    pltpu.sync_copy(rows_v.at[0, pl.ds(0, STRIPE - SK)],
                    c_sh.at[pl.ds(row0 + SK, STRIPE - SK)])
    plsc.subcore_barrier()

    lm = lm_v[...]

    def _p1(j, carry):
        off = e1 + j * SK
        pltpu.sync_copy(src_hbm.at[pl.ds(off, SK)], srcc_v)
        pltpu.sync_copy(dst_hbm.at[pl.ds(off, SK)], dstc_v)
        for g in range(GS):
            sl = pl.ds(g * 16, 16)
            d16 = dstc_v[sl]
            s16 = srcc_v[sl]
            t = plsc.load_gather(td_v, [d16]) + plsc.load_gather(ts_v, [s16])
            t = jnp.where(t >= 0.0, t, 0.01 * t)
            ex = jnp.exp(t - lm)
            a_v[0, pl.ds(g * 16, 16)] = ex
            didx2[0, g // Q, pl.ds((g % Q) * 16, 16)] = d16
        descs = [pltpu.async_copy(a_v.at[0, pl.ds(q * KC, KC)],
                                  den_sh.at[didx2.at[0, q]], sem_s, add=True)
                 for q in range(Q)]
        for dsc in descs:
            dsc.wait()
        return carry
    lax.fori_loop(0, NSK, _p1, 0)

    plsc.subcore_barrier()
    pltpu.sync_copy(den_sh.at[pl.ds(0, N)], den_v)

    def _prep(j, b):
        off = e1 + j * SK
        pltpu.sync_copy(src_hbm.at[pl.ds(off, SK)], srcc_v)
        pltpu.sync_copy(dst_hbm.at[pl.ds(off, SK)], dstc_v)
        for g in range(GS):
            sl = pl.ds(g * 16, 16)
            d16 = dstc_v[sl]
            s16 = srcc_v[sl]
            t = plsc.load_gather(td_v, [d16]) + plsc.load_gather(ts_v, [s16])
            t = jnp.where(t >= 0.0, t, 0.01 * t)
            ex = jnp.exp(t - lm)
            den = plsc.load_gather(den_v, [d16])
            a_v[b, pl.ds(g * 16, 16)] = ex / den
            sidx2[b, g // Q, pl.ds((g % Q) * 16, 16)] = s16
            didx2[b, g // Q, pl.ds((g % Q) * 16, 16)] = d16

    def _fire_gather(b):
        for q in range(Q):
            pltpu.async_copy(hv_hbm.at[c].at[sidx2.at[b, q]],
                             rows_v.at[b, pl.ds(q * KC, KC)], sem_g)

    def _drain_gather(b):
        for q in range(Q):
            pltpu.make_async_copy(hv_hbm.at[c].at[sidx2.at[b, q]],
                                  rows_v.at[b, pl.ds(q * KC, KC)], sem_g).wait()

    def _fire_scatter(b):
        for q in range(Q):
            for x in range(KC // 16):
                sdidx2[b, q, pl.ds(x * 16, 16)] = didx2[b, q, pl.ds(x * 16, 16)]
        for q in range(Q):
            pltpu.async_copy(rows_v.at[b, pl.ds(q * KC, KC)],
                             c_sh.at[sdidx2.at[b, q]], sem_s, add=True)

    def _drain_scatter(b):
        for q in range(Q):
            pltpu.make_async_copy(rows_v.at[b, pl.ds(q * KC, KC)],
                                  c_sh.at[sdidx2.at[b, q]], sem_s).wait()

    def _scale_buf(b):
        def _scale(e, carry):
            ae = plsc.load_gather(a_v.at[b], [jnp.zeros((16,), jnp.int32) + e])
            for r in range(DH // 16):
                sl = pl.ds(r * 16, 16)
                rows_v[b, e, sl] = rows_v[b, e, sl] * ae
            return carry
        lax.fori_loop(0, SK, _scale, 0)

    NPAIR = NSK // 2

    def _p2(jj, carry):
        j0 = jj * 2

        @pl.when(jj > 0)
        def _():
            _drain_scatter(1)
        _prep(j0 + 1, 1)
        _fire_gather(1)
        _drain_gather(0)
        _scale_buf(0)
        _fire_scatter(0)

        @pl.when(jj < NPAIR - 1)
        def _():
            _prep(j0 + 2, 0)
        _drain_gather(1)
        _scale_buf(1)
        _drain_scatter(0)

        @pl.when(jj < NPAIR - 1)
        def _():
            _fire_gather(0)
        _fire_scatter(1)
        return carry

    _prep(0, 0)
    _fire_gather(0)
    lax.fori_loop(0, NPAIR, _p2, 0)
    _drain_scatter(1)

    plsc.subcore_barrier()

    @pl.when(s < NS - 1)
    def _():
        pltpu.sync_copy(c_sh.at[pl.ds(row0, STRIPE)],
                        out_hbm.at[c, pl.ds(row0, STRIPE)])

    @pl.when(s == NS - 1)
    def _():
        pltpu.sync_copy(c_sh.at[pl.ds(row0, N - (NS - 1) * STRIPE)],
                        out_hbm.at[c, pl.ds(row0, N - (NS - 1) * STRIPE)])


def _post_body(cp_ref, nf_ref, w1c_ref, w1n_ref, b1_ref, w2_ref, b2_ref,
               g_ref, bt_ref, out_ref):
    csum = jnp.concatenate([cp_ref[0], cp_ref[1]], axis=1)
    ctx = jnp.where(csum > 0.0, csum, jnp.exp(jnp.minimum(csum, 0.0)) - 1.0)
    nf = nf_ref[...]
    h = (jnp.dot(ctx, w1c_ref[...], preferred_element_type=jnp.float32)
         + jnp.dot(nf, w1n_ref[...], preferred_element_type=jnp.float32)
         + b1_ref[...])
    h = jnp.maximum(h, 0.0)
    o = jnp.dot(h, w2_ref[...], preferred_element_type=jnp.float32) + b2_ref[...]
    o = jnp.maximum(o, 0.0)
    mean = jnp.mean(o, axis=0, keepdims=True)
    var = jnp.mean((o - mean) ** 2, axis=0, keepdims=True)
    out_ref[...] = (o - mean) * (g_ref[...] * lax.rsqrt(var + 1e-5)) + bt_ref[...]


def kernel(node_feats, edge_index, W_e, b_e, W_pn, b_pn, W1, b1, W2, b2,
           gamma, beta):
    f32 = jnp.float32
    hv, td, ts, lm = pl.pallas_call(
        _pre_body,
        out_shape=[
            jax.ShapeDtypeStruct((NC, N, DH), f32),
            jax.ShapeDtypeStruct((N, 1), f32),
            jax.ShapeDtypeStruct((N, 1), f32),
            jax.ShapeDtypeStruct((8, 128), f32),
        ],
    )(node_feats, W_e, W_pn, b_pn.reshape(1, D), b_e.reshape(1, 1))

    cparts = _sc_main(td.reshape(N), ts.reshape(N), lm[0, :16], hv,
                      edge_index[0], edge_index[1])

    out = pl.pallas_call(
        _post_body,
        out_shape=jax.ShapeDtypeStruct((N, D), f32),
    )(cparts, node_feats, W1[:D], W1[D:], b1.reshape(1, D), W2,
      b2.reshape(1, D), gamma.reshape(1, D), beta.reshape(1, D))
    return out


# speedup vs baseline: 1.1556x; 1.1556x over previous
import functools

import jax
import jax.numpy as jnp
from jax import lax
from jax.experimental import pallas as pl
from jax.experimental.pallas import tpu as pltpu
from jax.experimental.pallas import tpu_sc as plsc

N = 10000
E = 320000
D = 128
DH = D // 2
NC = 2
NS = 16
N2 = 10240
STRIPE = N2 // NS
EP = E // NS
KC = 80
Q = 5
SK = KC * Q
NSK = EP // SK
GS = SK // 16


def _pre_body(nf_ref, we_ref, wpn_ref, bpn_ref, be_ref,
              hv_ref, td_ref, ts_ref, lm_ref):
    nf = nf_ref[...]
    hv = (jnp.dot(nf, wpn_ref[...], preferred_element_type=jnp.float32)
          + bpn_ref[...])
    hv_ref[0] = hv[:, :DH]
    hv_ref[1] = hv[:, DH:]
    td = jnp.dot(nf, we_ref[:D, :], preferred_element_type=jnp.float32) + be_ref[0, 0]
    ts = jnp.dot(nf, we_ref[D:, :], preferred_element_type=jnp.float32)
    td_ref[...] = td
    ts_ref[...] = ts
    ub = jnp.max(td) + jnp.max(ts)
    lm = jnp.where(ub >= 0.0, ub, 0.01 * ub)
    lm_ref[...] = jnp.full((8, 128), lm, jnp.float32)


_MESH = plsc.VectorSubcoreMesh(core_axis_name="c", subcore_axis_name="s",
                               num_cores=NC, num_subcores=NS)


@functools.partial(
    pl.kernel,
    out_type=(jax.ShapeDtypeStruct((NC, N, DH), jnp.float32),
              jax.ShapeDtypeStruct((NC, N), jnp.float32)),
    mesh=_MESH,
    compiler_params=pltpu.CompilerParams(needs_layout_passes=False,
                                         use_tc_tiling_on_sc=False),
    scratch_types=[
        pltpu.VMEM((N,), jnp.float32),
        pltpu.VMEM((N,), jnp.float32),
        pltpu.VMEM((16,), jnp.float32),
        pltpu.VMEM((SK,), jnp.int32),
        pltpu.VMEM((SK,), jnp.int32),
        pltpu.VMEM((STRIPE,), jnp.float32),
        pltpu.VMEM((2, SK), jnp.float32),
        pltpu.VMEM((2, SK), jnp.float32),
        pltpu.VMEM((2, Q, KC), jnp.int32),
        pltpu.VMEM((2, Q, KC), jnp.int32),
        pltpu.VMEM((2, Q, KC), jnp.int32),
        pltpu.VMEM((2, SK, DH), jnp.float32),
        pltpu.VMEM_SHARED((N2,), jnp.float32),
        pltpu.VMEM_SHARED((N2, DH), jnp.float32),
        pltpu.SemaphoreType.DMA,
        pltpu.SemaphoreType.DMA,
    ],
)
def _sc_main(td_hbm, ts_hbm, lm_hbm, hv_hbm, src_hbm, dst_hbm, out_hbm, den_hbm,
             td_v, ts_v, lm_v, srcc_v, dstc_v, zro_v,
             a_v, sa_v, sidx2, didx2, sdidx2, rows_v, den_sh, c_sh, sem_g, sem_s):
    c = lax.axis_index("c")
    s = lax.axis_index("s")
    zf = jnp.zeros((16,), jnp.float32)

    pltpu.sync_copy(td_hbm, td_v)
    pltpu.sync_copy(ts_hbm, ts_v)
    pltpu.sync_copy(lm_hbm, lm_v)
    e1 = s * EP

    def _z_rows(i, carry):
        for r in range(DH // 16):
            rows_v[0, i, pl.ds(r * 16, 16)] = zf
        return carry
    lax.fori_loop(0, SK, _z_rows, 0)

    def _z(i, carry):
        zro_v[pl.ds(i * 16, 16)] = zf
        return carry
    lax.fori_loop(0, STRIPE // 16, _z, 0)

    row0 = s * STRIPE
    pltpu.sync_copy(zro_v, den_sh.at[pl.ds(row0, STRIPE)])
    pltpu.sync_copy(rows_v.at[0], c_sh.at[pl.ds(row0, SK)])
    pltpu.sync_copy(rows_v.at[0, pl.ds(0, STRIPE - SK)],
                    c_sh.at[pl.ds(row0 + SK, STRIPE - SK)])
    plsc.subcore_barrier()

    lm = lm_v[...]

    def _prep(j, b):
        off = e1 + j * SK
        pltpu.sync_copy(src_hbm.at[pl.ds(off, SK)], srcc_v)
        pltpu.sync_copy(dst_hbm.at[pl.ds(off, SK)], dstc_v)
        for g in range(GS):
            sl = pl.ds(g * 16, 16)
            d16 = dstc_v[sl]
            s16 = srcc_v[sl]
            t = plsc.load_gather(td_v, [d16]) + plsc.load_gather(ts_v, [s16])
            t = jnp.where(t >= 0.0, t, 0.01 * t)
            ex = jnp.exp(t - lm)
            a_v[b, pl.ds(g * 16, 16)] = ex
            sidx2[b, g // Q, pl.ds((g % Q) * 16, 16)] = s16
            didx2[b, g // Q, pl.ds((g % Q) * 16, 16)] = d16

    def _fire_gather(b):
        for q in range(Q):
            pltpu.async_copy(hv_hbm.at[c].at[sidx2.at[b, q]],
                             rows_v.at[b, pl.ds(q * KC, KC)], sem_g)

    def _drain_gather(b):
        for q in range(Q):
            pltpu.make_async_copy(hv_hbm.at[c].at[sidx2.at[b, q]],
                                  rows_v.at[b, pl.ds(q * KC, KC)], sem_g).wait()

    def _fire_scatter(b):
        for q in range(Q):
            for x in range(KC // 16):
                sdidx2[b, q, pl.ds(x * 16, 16)] = didx2[b, q, pl.ds(x * 16, 16)]
                sa_v[b, pl.ds(q * KC + x * 16, 16)] = a_v[b, pl.ds(q * KC + x * 16, 16)]
        for q in range(Q):
            pltpu.async_copy(rows_v.at[b, pl.ds(q * KC, KC)],
                             c_sh.at[sdidx2.at[b, q]], sem_s, add=True)
            pltpu.async_copy(sa_v.at[b, pl.ds(q * KC, KC)],
                             den_sh.at[sdidx2.at[b, q]], sem_s, add=True)

    def _drain_scatter(b):
        for q in range(Q):
            pltpu.make_async_copy(rows_v.at[b, pl.ds(q * KC, KC)],
                                  c_sh.at[sdidx2.at[b, q]], sem_s).wait()
            pltpu.make_async_copy(sa_v.at[b, pl.ds(q * KC, KC)],
                                  den_sh.at[sdidx2.at[b, q]], sem_s).wait()

    def _scale_buf(b):
        def _scale(e, carry):
            ae = plsc.load_gather(a_v.at[b], [jnp.zeros((16,), jnp.int32) + e])
            for r in range(DH // 16):
                sl = pl.ds(r * 16, 16)
                rows_v[b, e, sl] = rows_v[b, e, sl] * ae
            return carry
        lax.fori_loop(0, SK, _scale, 0)

    NPAIR = NSK // 2

    def _p2(jj, carry):
        j0 = jj * 2

        @pl.when(jj > 0)
        def _():
            _drain_scatter(1)
        _prep(j0 + 1, 1)
        _fire_gather(1)
        _drain_gather(0)
        _scale_buf(0)
        _fire_scatter(0)

        @pl.when(jj < NPAIR - 1)
        def _():
            _prep(j0 + 2, 0)
        _drain_gather(1)
        _scale_buf(1)
        _drain_scatter(0)

        @pl.when(jj < NPAIR - 1)
        def _():
            _fire_gather(0)
        _fire_scatter(1)
        return carry

    _prep(0, 0)
    _fire_gather(0)
    lax.fori_loop(0, NPAIR, _p2, 0)
    _drain_scatter(1)

    plsc.subcore_barrier()

    @pl.when(s < NS - 1)
    def _():
        pltpu.sync_copy(c_sh.at[pl.ds(row0, STRIPE)],
                        out_hbm.at[c, pl.ds(row0, STRIPE)])
        pltpu.sync_copy(den_sh.at[pl.ds(row0, STRIPE)],
                        den_hbm.at[c, pl.ds(row0, STRIPE)])

    @pl.when(s == NS - 1)
    def _():
        pltpu.sync_copy(c_sh.at[pl.ds(row0, N - (NS - 1) * STRIPE)],
                        out_hbm.at[c, pl.ds(row0, N - (NS - 1) * STRIPE)])
        pltpu.sync_copy(den_sh.at[pl.ds(row0, N - (NS - 1) * STRIPE)],
                        den_hbm.at[c, pl.ds(row0, N - (NS - 1) * STRIPE)])


def _post_body(cp_ref, den_ref, nf_ref, w1c_ref, w1n_ref, b1_ref, w2_ref, b2_ref,
               g_ref, bt_ref, out_ref):
    den = den_ref[0]
    recip = jnp.where(den > 0.0, 1.0 / den, 0.0)
    csum = jnp.concatenate([cp_ref[0], cp_ref[1]], axis=1) * recip
    ctx = jnp.where(csum > 0.0, csum, jnp.exp(jnp.minimum(csum, 0.0)) - 1.0)
    nf = nf_ref[...]
    h = (jnp.dot(ctx, w1c_ref[...], preferred_element_type=jnp.float32)
         + jnp.dot(nf, w1n_ref[...], preferred_element_type=jnp.float32)
         + b1_ref[...])
    h = jnp.maximum(h, 0.0)
    o = jnp.dot(h, w2_ref[...], preferred_element_type=jnp.float32) + b2_ref[...]
    o = jnp.maximum(o, 0.0)
    mean = jnp.mean(o, axis=0, keepdims=True)
    var = jnp.mean((o - mean) ** 2, axis=0, keepdims=True)
    out_ref[...] = (o - mean) * (g_ref[...] * lax.rsqrt(var + 1e-5)) + bt_ref[...]


def kernel(node_feats, edge_index, W_e, b_e, W_pn, b_pn, W1, b1, W2, b2,
           gamma, beta):
    f32 = jnp.float32
    hv, td, ts, lm = pl.pallas_call(
        _pre_body,
        out_shape=[
            jax.ShapeDtypeStruct((NC, N, DH), f32),
            jax.ShapeDtypeStruct((N, 1), f32),
            jax.ShapeDtypeStruct((N, 1), f32),
            jax.ShapeDtypeStruct((8, 128), f32),
        ],
    )(node_feats, W_e, W_pn, b_pn.reshape(1, D), b_e.reshape(1, 1))

    cparts, denp = _sc_main(td.reshape(N), ts.reshape(N), lm[0, :16], hv,
                            edge_index[0], edge_index[1])

    out = pl.pallas_call(
        _post_body,
        out_shape=jax.ShapeDtypeStruct((N, D), f32),
    )(cparts, denp.reshape(NC, N, 1), node_feats, W1[:D], W1[D:],
      b1.reshape(1, D), W2, b2.reshape(1, D), gamma.reshape(1, D),
      beta.reshape(1, D))
    return out


# speedup vs baseline: 1.5470x; 1.3387x over previous
import functools

import jax
import jax.numpy as jnp
from jax import lax
from jax.experimental import pallas as pl
from jax.experimental.pallas import tpu as pltpu
from jax.experimental.pallas import tpu_sc as plsc

N = 10000
E = 320000
D = 128
DH = D // 2
NC = 2
NS = 16
N2 = 10240
STRIPE = N2 // NS
EP = E // NS
KC = 80
Q = 5
SK = KC * Q
NSK = EP // SK
GS = SK // 16


def _pre_body(nf_ref, we_ref, wpn_ref, bpn_ref, be_ref,
              hv_ref, td_ref, ts_ref, lm_ref):
    nf = nf_ref[...]
    hv = (jnp.dot(nf, wpn_ref[...], preferred_element_type=jnp.float32)
          + bpn_ref[...])
    hv_ref[0] = hv[:, :DH]
    hv_ref[1] = hv[:, DH:]
    td = jnp.dot(nf, we_ref[:D, :], preferred_element_type=jnp.float32) + be_ref[0, 0]
    ts = jnp.dot(nf, we_ref[D:, :], preferred_element_type=jnp.float32)
    td_ref[...] = td
    ts_ref[...] = ts
    ub = jnp.max(td) + jnp.max(ts)
    lm = jnp.where(ub >= 0.0, ub, 0.01 * ub)
    lm_ref[...] = jnp.full((8, 128), lm, jnp.float32)


_MESH = plsc.VectorSubcoreMesh(core_axis_name="c", subcore_axis_name="s",
                               num_cores=NC, num_subcores=NS)


@functools.partial(
    pl.kernel,
    out_type=(jax.ShapeDtypeStruct((NC, N, DH), jnp.float32),
              jax.ShapeDtypeStruct((NC, N), jnp.float32)),
    mesh=_MESH,
    compiler_params=pltpu.CompilerParams(needs_layout_passes=False,
                                         use_tc_tiling_on_sc=False),
    scratch_types=[
        pltpu.VMEM((N,), jnp.float32),
        pltpu.VMEM((N,), jnp.float32),
        pltpu.VMEM((16,), jnp.float32),
        pltpu.VMEM((SK,), jnp.int32),
        pltpu.VMEM((SK,), jnp.int32),
        pltpu.VMEM((STRIPE,), jnp.float32),
        pltpu.VMEM((2, SK), jnp.float32),
        pltpu.VMEM((2, SK), jnp.float32),
        pltpu.VMEM((2, Q, KC), jnp.int32),
        pltpu.VMEM((2, Q, KC), jnp.int32),
        pltpu.VMEM((2, Q, KC), jnp.int32),
        pltpu.VMEM((2, SK, DH), jnp.float32),
        pltpu.VMEM_SHARED((N2,), jnp.float32),
        pltpu.VMEM_SHARED((N2, DH), jnp.float32),
        pltpu.SemaphoreType.DMA,
        pltpu.SemaphoreType.DMA,
    ],
)
def _sc_main(td_hbm, ts_hbm, lm_hbm, hv_hbm, src_hbm, dst_hbm, out_hbm, den_hbm,
             td_v, ts_v, lm_v, srcc_v, dstc_v, zro_v,
             a_v, sa_v, sidx2, didx2, sdidx2, rows_v, den_sh, c_sh, sem_g, sem_s):
    c = lax.axis_index("c")
    s = lax.axis_index("s")
    zf = jnp.zeros((16,), jnp.float32)

    pltpu.sync_copy(td_hbm, td_v)
    pltpu.sync_copy(ts_hbm, ts_v)
    pltpu.sync_copy(lm_hbm, lm_v)
    e1 = s * EP

    def _z_rows(i, carry):
        for r in range(DH // 16):
            rows_v[0, i, pl.ds(r * 16, 16)] = zf
        return carry
    lax.fori_loop(0, SK, _z_rows, 0)

    def _z(i, carry):
        zro_v[pl.ds(i * 16, 16)] = zf
        return carry
    lax.fori_loop(0, STRIPE // 16, _z, 0)

    row0 = s * STRIPE
    pltpu.sync_copy(zro_v, den_sh.at[pl.ds(row0, STRIPE)])
    pltpu.sync_copy(rows_v.at[0], c_sh.at[pl.ds(row0, SK)])
    pltpu.sync_copy(rows_v.at[0, pl.ds(0, STRIPE - SK)],
                    c_sh.at[pl.ds(row0 + SK, STRIPE - SK)])
    plsc.subcore_barrier()

    lm = lm_v[...]

    def _prep(j, b):
        off = e1 + j * SK
        pltpu.sync_copy(src_hbm.at[pl.ds(off, SK)], srcc_v)
        pltpu.sync_copy(dst_hbm.at[pl.ds(off, SK)], dstc_v)
        for g in range(GS):
            sl = pl.ds(g * 16, 16)
            d16 = dstc_v[sl]
            s16 = srcc_v[sl]
            t = plsc.load_gather(td_v, [d16]) + plsc.load_gather(ts_v, [s16])
            t = jnp.where(t >= 0.0, t, 0.01 * t)
            ex = jnp.exp(t - lm)
            a_v[b, pl.ds(g * 16, 16)] = ex
            sidx2[b, g // Q, pl.ds((g % Q) * 16, 16)] = s16
            didx2[b, g // Q, pl.ds((g % Q) * 16, 16)] = d16

    def _fire_gather(b):
        for q in range(Q):
            pltpu.async_copy(hv_hbm.at[c].at[sidx2.at[b, q]],
                             rows_v.at[b, pl.ds(q * KC, KC)], sem_g)

    def _drain_gather(b):
        for q in range(Q):
            pltpu.make_async_copy(hv_hbm.at[c].at[sidx2.at[b, q]],
                                  rows_v.at[b, pl.ds(q * KC, KC)], sem_g).wait()

    def _fire_scatter(b):
        for q in range(Q):
            for x in range(KC // 16):
                sdidx2[b, q, pl.ds(x * 16, 16)] = didx2[b, q, pl.ds(x * 16, 16)]
                sa_v[b, pl.ds(q * KC + x * 16, 16)] = a_v[b, pl.ds(q * KC + x * 16, 16)]
        for q in range(Q):
            pltpu.async_copy(rows_v.at[b, pl.ds(q * KC, KC)],
                             c_sh.at[sdidx2.at[b, q]], sem_s, add=True)
            pltpu.async_copy(sa_v.at[b, pl.ds(q * KC, KC)],
                             den_sh.at[sdidx2.at[b, q]], sem_s, add=True)

    def _drain_scatter(b):
        for q in range(Q):
            pltpu.make_async_copy(rows_v.at[b, pl.ds(q * KC, KC)],
                                  c_sh.at[sdidx2.at[b, q]], sem_s).wait()
            pltpu.make_async_copy(sa_v.at[b, pl.ds(q * KC, KC)],
                                  den_sh.at[sdidx2.at[b, q]], sem_s).wait()

    def _scale_buf(b):
        @plsc.parallel_loop(0, SK, step=1, unroll=4)
        def _scale(e):
            ae = plsc.load_gather(a_v.at[b], [jnp.zeros((16,), jnp.int32) + e])
            for r in range(DH // 16):
                sl = pl.ds(r * 16, 16)
                rows_v[b, e, sl] = rows_v[b, e, sl] * ae

    NPAIR = NSK // 2

    def _p2(jj, carry):
        j0 = jj * 2

        @pl.when(jj > 0)
        def _():
            _drain_scatter(1)
        _prep(j0 + 1, 1)
        _fire_gather(1)
        _drain_gather(0)
        _scale_buf(0)
        _fire_scatter(0)

        @pl.when(jj < NPAIR - 1)
        def _():
            _prep(j0 + 2, 0)
        _drain_gather(1)
        _scale_buf(1)
        _drain_scatter(0)

        @pl.when(jj < NPAIR - 1)
        def _():
            _fire_gather(0)
        _fire_scatter(1)
        return carry

    _prep(0, 0)
    _fire_gather(0)
    lax.fori_loop(0, NPAIR, _p2, 0)
    _drain_scatter(1)

    plsc.subcore_barrier()

    @pl.when(s < NS - 1)
    def _():
        pltpu.sync_copy(c_sh.at[pl.ds(row0, STRIPE)],
                        out_hbm.at[c, pl.ds(row0, STRIPE)])
        pltpu.sync_copy(den_sh.at[pl.ds(row0, STRIPE)],
                        den_hbm.at[c, pl.ds(row0, STRIPE)])

    @pl.when(s == NS - 1)
    def _():
        pltpu.sync_copy(c_sh.at[pl.ds(row0, N - (NS - 1) * STRIPE)],
                        out_hbm.at[c, pl.ds(row0, N - (NS - 1) * STRIPE)])
        pltpu.sync_copy(den_sh.at[pl.ds(row0, N - (NS - 1) * STRIPE)],
                        den_hbm.at[c, pl.ds(row0, N - (NS - 1) * STRIPE)])


def _post_body(cp_ref, den_ref, nf_ref, w1c_ref, w1n_ref, b1_ref, w2_ref, b2_ref,
               g_ref, bt_ref, out_ref):
    den = den_ref[0]
    recip = jnp.where(den > 0.0, 1.0 / den, 0.0)
    csum = jnp.concatenate([cp_ref[0], cp_ref[1]], axis=1) * recip
    ctx = jnp.where(csum > 0.0, csum, jnp.exp(jnp.minimum(csum, 0.0)) - 1.0)
    nf = nf_ref[...]
    h = (jnp.dot(ctx, w1c_ref[...], preferred_element_type=jnp.float32)
         + jnp.dot(nf, w1n_ref[...], preferred_element_type=jnp.float32)
         + b1_ref[...])
    h = jnp.maximum(h, 0.0)
    o = jnp.dot(h, w2_ref[...], preferred_element_type=jnp.float32) + b2_ref[...]
    o = jnp.maximum(o, 0.0)
    mean = jnp.mean(o, axis=0, keepdims=True)
    var = jnp.mean((o - mean) ** 2, axis=0, keepdims=True)
    out_ref[...] = (o - mean) * (g_ref[...] * lax.rsqrt(var + 1e-5)) + bt_ref[...]


def kernel(node_feats, edge_index, W_e, b_e, W_pn, b_pn, W1, b1, W2, b2,
           gamma, beta):
    f32 = jnp.float32
    hv, td, ts, lm = pl.pallas_call(
        _pre_body,
        out_shape=[
            jax.ShapeDtypeStruct((NC, N, DH), f32),
            jax.ShapeDtypeStruct((N, 1), f32),
            jax.ShapeDtypeStruct((N, 1), f32),
            jax.ShapeDtypeStruct((8, 128), f32),
        ],
    )(node_feats, W_e, W_pn, b_pn.reshape(1, D), b_e.reshape(1, 1))

    cparts, denp = _sc_main(td.reshape(N), ts.reshape(N), lm[0, :16], hv,
                            edge_index[0], edge_index[1])

    out = pl.pallas_call(
        _post_body,
        out_shape=jax.ShapeDtypeStruct((N, D), f32),
    )(cparts, denp.reshape(NC, N, 1), node_feats, W1[:D], W1[D:],
      b1.reshape(1, D), W2, b2.reshape(1, D), gamma.reshape(1, D),
      beta.reshape(1, D))
    return out


# speedup vs baseline: 1.7579x; 1.1363x over previous
import functools

import jax
import jax.numpy as jnp
from jax import lax
from jax.experimental import pallas as pl
from jax.experimental.pallas import tpu as pltpu
from jax.experimental.pallas import tpu_sc as plsc

N = 10000
E = 320000
D = 128
DH = D // 2
NC = 2
NS = 16
N2 = 10240
STRIPE = N2 // NS
EP = E // NS
KC = 80
Q = 5
SK = KC * Q
NSK = EP // SK
GS = SK // 16


def _pre_body(nf_ref, we_ref, wpn_ref, bpn_ref, be_ref,
              hv_ref, td_ref, ts_ref, lm_ref):
    nf = nf_ref[...]
    hv = (jnp.dot(nf, wpn_ref[...], preferred_element_type=jnp.float32)
          + bpn_ref[...])
    hv_ref[0] = hv[:, :DH]
    hv_ref[1] = hv[:, DH:]
    td = jnp.dot(nf, we_ref[:D, :], preferred_element_type=jnp.float32) + be_ref[0, 0]
    ts = jnp.dot(nf, we_ref[D:, :], preferred_element_type=jnp.float32)
    td_ref[...] = td
    ts_ref[...] = ts
    ub = jnp.max(td) + jnp.max(ts)
    lm = jnp.where(ub >= 0.0, ub, 0.01 * ub)
    lm_ref[...] = jnp.full((8, 128), lm, jnp.float32)


_MESH = plsc.VectorSubcoreMesh(core_axis_name="c", subcore_axis_name="s",
                               num_cores=NC, num_subcores=NS)


@functools.partial(
    pl.kernel,
    out_type=(jax.ShapeDtypeStruct((NC, N, DH), jnp.float32),
              jax.ShapeDtypeStruct((NC, N), jnp.float32)),
    mesh=_MESH,
    compiler_params=pltpu.CompilerParams(needs_layout_passes=False,
                                         use_tc_tiling_on_sc=False),
    scratch_types=[
        pltpu.VMEM((N,), jnp.float32),
        pltpu.VMEM((N,), jnp.float32),
        pltpu.VMEM((16,), jnp.float32),
        pltpu.VMEM((2, SK), jnp.int32),
        pltpu.VMEM((2, SK), jnp.int32),
        pltpu.VMEM((STRIPE,), jnp.float32),
        pltpu.VMEM((2, SK), jnp.float32),
        pltpu.VMEM((2, SK), jnp.float32),
        pltpu.VMEM((2, Q, KC), jnp.int32),
        pltpu.VMEM((2, Q, KC), jnp.int32),
        pltpu.VMEM((2, Q, KC), jnp.int32),
        pltpu.VMEM((2, SK, DH), jnp.float32),
        pltpu.VMEM_SHARED((N2,), jnp.float32),
        pltpu.VMEM_SHARED((N2, DH), jnp.float32),
        pltpu.SemaphoreType.DMA,
        pltpu.SemaphoreType.DMA,
        pltpu.SemaphoreType.DMA,
    ],
)
def _sc_main(td_hbm, ts_hbm, lm_hbm, hv_hbm, src_hbm, dst_hbm, out_hbm, den_hbm,
             td_v, ts_v, lm_v, srcc_v, dstc_v, zro_v,
             a_v, sa_v, sidx2, didx2, sdidx2, rows_v, den_sh, c_sh,
             sem_g, sem_s, sem_i):
    c = lax.axis_index("c")
    s = lax.axis_index("s")
    zf = jnp.zeros((16,), jnp.float32)

    pltpu.sync_copy(td_hbm, td_v)
    pltpu.sync_copy(ts_hbm, ts_v)
    pltpu.sync_copy(lm_hbm, lm_v)
    e1 = s * EP

    def _z_rows(i, carry):
        for r in range(DH // 16):
            rows_v[0, i, pl.ds(r * 16, 16)] = zf
        return carry
    lax.fori_loop(0, SK, _z_rows, 0)

    def _z(i, carry):
        zro_v[pl.ds(i * 16, 16)] = zf
        return carry
    lax.fori_loop(0, STRIPE // 16, _z, 0)

    row0 = s * STRIPE
    pltpu.sync_copy(zro_v, den_sh.at[pl.ds(row0, STRIPE)])
    pltpu.sync_copy(rows_v.at[0], c_sh.at[pl.ds(row0, SK)])
    pltpu.sync_copy(rows_v.at[0, pl.ds(0, STRIPE - SK)],
                    c_sh.at[pl.ds(row0 + SK, STRIPE - SK)])
    plsc.subcore_barrier()

    lm = lm_v[...]

    def _fetch(j, b):
        off = e1 + j * SK
        pltpu.async_copy(src_hbm.at[pl.ds(off, SK)], srcc_v.at[b], sem_i)
        pltpu.async_copy(dst_hbm.at[pl.ds(off, SK)], dstc_v.at[b], sem_i)

    def _prep(j, b):
        off = e1 + j * SK
        pltpu.make_async_copy(src_hbm.at[pl.ds(off, SK)], srcc_v.at[b], sem_i).wait()
        pltpu.make_async_copy(dst_hbm.at[pl.ds(off, SK)], dstc_v.at[b], sem_i).wait()
        for g in range(GS):
            sl = pl.ds(g * 16, 16)
            d16 = dstc_v[b, sl]
            s16 = srcc_v[b, sl]
            t = plsc.load_gather(td_v, [d16]) + plsc.load_gather(ts_v, [s16])
            t = jnp.where(t >= 0.0, t, 0.01 * t)
            ex = jnp.exp(t - lm)
            a_v[b, pl.ds(g * 16, 16)] = ex
            sidx2[b, g // Q, pl.ds((g % Q) * 16, 16)] = s16
            didx2[b, g // Q, pl.ds((g % Q) * 16, 16)] = d16

    def _fire_gather(b):
        for q in range(Q):
            pltpu.async_copy(hv_hbm.at[c].at[sidx2.at[b, q]],
                             rows_v.at[b, pl.ds(q * KC, KC)], sem_g)

    def _drain_gather(b):
        for q in range(Q):
            pltpu.make_async_copy(hv_hbm.at[c].at[sidx2.at[b, q]],
                                  rows_v.at[b, pl.ds(q * KC, KC)], sem_g).wait()

    def _fire_scatter(b):
        for q in range(Q):
            for x in range(KC // 16):
                sdidx2[b, q, pl.ds(x * 16, 16)] = didx2[b, q, pl.ds(x * 16, 16)]
                sa_v[b, pl.ds(q * KC + x * 16, 16)] = a_v[b, pl.ds(q * KC + x * 16, 16)]
        for q in range(Q):
            pltpu.async_copy(rows_v.at[b, pl.ds(q * KC, KC)],
                             c_sh.at[sdidx2.at[b, q]], sem_s, add=True)
            pltpu.async_copy(sa_v.at[b, pl.ds(q * KC, KC)],
                             den_sh.at[sdidx2.at[b, q]], sem_s, add=True)

    def _drain_scatter(b):
        for q in range(Q):
            pltpu.make_async_copy(rows_v.at[b, pl.ds(q * KC, KC)],
                                  c_sh.at[sdidx2.at[b, q]], sem_s).wait()
            pltpu.make_async_copy(sa_v.at[b, pl.ds(q * KC, KC)],
                                  den_sh.at[sdidx2.at[b, q]], sem_s).wait()

    def _scale_buf(b):
        @plsc.parallel_loop(0, SK, step=1, unroll=4)
        def _scale(e):
            ae = plsc.load_gather(a_v.at[b], [jnp.zeros((16,), jnp.int32) + e])
            for r in range(DH // 16):
                sl = pl.ds(r * 16, 16)
                rows_v[b, e, sl] = rows_v[b, e, sl] * ae

    NPAIR = NSK // 2

    def _p2(jj, carry):
        j0 = jj * 2

        @pl.when(jj > 0)
        def _():
            _drain_scatter(1)
        _prep(j0 + 1, 1)
        _fire_gather(1)

        @pl.when(jj < NPAIR - 1)
        def _():
            _fetch(j0 + 2, 0)
        _drain_gather(0)
        _scale_buf(0)
        _fire_scatter(0)

        @pl.when(jj < NPAIR - 1)
        def _():
            _prep(j0 + 2, 0)
        _drain_gather(1)
        _scale_buf(1)
        _drain_scatter(0)

        @pl.when(jj < NPAIR - 1)
        def _():
            _fire_gather(0)
            _fetch(j0 + 3, 1)
        _fire_scatter(1)
        return carry

    _fetch(0, 0)
    _fetch(1, 1)
    _prep(0, 0)
    _fire_gather(0)
    lax.fori_loop(0, NPAIR, _p2, 0)
    _drain_scatter(1)

    plsc.subcore_barrier()

    @pl.when(s < NS - 1)
    def _():
        pltpu.sync_copy(c_sh.at[pl.ds(row0, STRIPE)],
                        out_hbm.at[c, pl.ds(row0, STRIPE)])
        pltpu.sync_copy(den_sh.at[pl.ds(row0, STRIPE)],
                        den_hbm.at[c, pl.ds(row0, STRIPE)])

    @pl.when(s == NS - 1)
    def _():
        pltpu.sync_copy(c_sh.at[pl.ds(row0, N - (NS - 1) * STRIPE)],
                        out_hbm.at[c, pl.ds(row0, N - (NS - 1) * STRIPE)])
        pltpu.sync_copy(den_sh.at[pl.ds(row0, N - (NS - 1) * STRIPE)],
                        den_hbm.at[c, pl.ds(row0, N - (NS - 1) * STRIPE)])


def _post_body(cp_ref, den_ref, nf_ref, w1c_ref, w1n_ref, b1_ref, w2_ref, b2_ref,
               g_ref, bt_ref, out_ref):
    den = den_ref[0]
    recip = jnp.where(den > 0.0, 1.0 / den, 0.0)
    csum = jnp.concatenate([cp_ref[0], cp_ref[1]], axis=1) * recip
    ctx = jnp.where(csum > 0.0, csum, jnp.exp(jnp.minimum(csum, 0.0)) - 1.0)
    nf = nf_ref[...]
    h = (jnp.dot(ctx, w1c_ref[...], preferred_element_type=jnp.float32)
         + jnp.dot(nf, w1n_ref[...], preferred_element_type=jnp.float32)
         + b1_ref[...])
    h = jnp.maximum(h, 0.0)
    o = jnp.dot(h, w2_ref[...], preferred_element_type=jnp.float32) + b2_ref[...]
    o = jnp.maximum(o, 0.0)
    mean = jnp.mean(o, axis=0, keepdims=True)
    var = jnp.mean((o - mean) ** 2, axis=0, keepdims=True)
    out_ref[...] = (o - mean) * (g_ref[...] * lax.rsqrt(var + 1e-5)) + bt_ref[...]


def kernel(node_feats, edge_index, W_e, b_e, W_pn, b_pn, W1, b1, W2, b2,
           gamma, beta):
    f32 = jnp.float32
    hv, td, ts, lm = pl.pallas_call(
        _pre_body,
        out_shape=[
            jax.ShapeDtypeStruct((NC, N, DH), f32),
            jax.ShapeDtypeStruct((N, 1), f32),
            jax.ShapeDtypeStruct((N, 1), f32),
            jax.ShapeDtypeStruct((8, 128), f32),
        ],
    )(node_feats, W_e, W_pn, b_pn.reshape(1, D), b_e.reshape(1, 1))

    cparts, denp = _sc_main(td.reshape(N), ts.reshape(N), lm[0, :16], hv,
                            edge_index[0], edge_index[1])

    out = pl.pallas_call(
        _post_body,
        out_shape=jax.ShapeDtypeStruct((N, D), f32),
    )(cparts, denp.reshape(NC, N, 1), node_feats, W1[:D], W1[D:],
      b1.reshape(1, D), W2, b2.reshape(1, D), gamma.reshape(1, D),
      beta.reshape(1, D))
    return out


# speedup vs baseline: 1.8397x; 1.0465x over previous
import functools

import jax
import jax.numpy as jnp
from jax import lax
from jax.experimental import pallas as pl
from jax.experimental.pallas import tpu as pltpu
from jax.experimental.pallas import tpu_sc as plsc

N = 10000
E = 320000
D = 128
DH = D // 2
NC = 2
NS = 16
N2 = 10240
STRIPE = N2 // NS
EP = E // NS
KC = 80
Q = 5
SK = KC * Q
NSK = EP // SK
GS = SK // 16


def _pre_body(nf_ref, we_ref, wpn_ref, bpn_ref, be_ref,
              hv_ref, td_ref, ts_ref, lm_ref):
    nf = nf_ref[...]
    hv = (jnp.dot(nf, wpn_ref[...], preferred_element_type=jnp.float32)
          + bpn_ref[...])
    hv_ref[0] = hv[:, :DH]
    hv_ref[1] = hv[:, DH:]
    td = jnp.dot(nf, we_ref[:D, :], preferred_element_type=jnp.float32) + be_ref[0, 0]
    ts = jnp.dot(nf, we_ref[D:, :], preferred_element_type=jnp.float32)
    td_ref[...] = td
    ts_ref[...] = ts
    ub = jnp.max(td) + jnp.max(ts)
    lm = jnp.where(ub >= 0.0, ub, 0.01 * ub)
    lm_ref[...] = jnp.full((1, 16), lm, jnp.float32)


_MESH = plsc.VectorSubcoreMesh(core_axis_name="c", subcore_axis_name="s",
                               num_cores=NC, num_subcores=NS)


@functools.partial(
    pl.kernel,
    out_type=(jax.ShapeDtypeStruct((NC, N, DH), jnp.float32),
              jax.ShapeDtypeStruct((NC, N), jnp.float32)),
    mesh=_MESH,
    compiler_params=pltpu.CompilerParams(needs_layout_passes=False,
                                         use_tc_tiling_on_sc=False),
    scratch_types=[
        pltpu.VMEM((N,), jnp.float32),
        pltpu.VMEM((N,), jnp.float32),
        pltpu.VMEM((16,), jnp.float32),
        pltpu.VMEM((2, SK), jnp.int32),
        pltpu.VMEM((2, SK), jnp.int32),
        pltpu.VMEM((STRIPE,), jnp.float32),
        pltpu.VMEM((2, SK), jnp.float32),
        pltpu.VMEM((2, SK), jnp.float32),
        pltpu.VMEM((2, Q, KC), jnp.int32),
        pltpu.VMEM((2, Q, KC), jnp.int32),
        pltpu.VMEM((2, Q, KC), jnp.int32),
        pltpu.VMEM((2, SK, DH), jnp.float32),
        pltpu.VMEM_SHARED((N2,), jnp.float32),
        pltpu.VMEM_SHARED((N2, DH), jnp.float32),
        pltpu.SemaphoreType.DMA,
        pltpu.SemaphoreType.DMA,
        pltpu.SemaphoreType.DMA,
    ],
)
def _sc_main(td_hbm, ts_hbm, lm_hbm, hv_hbm, ei_hbm, out_hbm, den_hbm,
             td_v, ts_v, lm_v, srcc_v, dstc_v, zro_v,
             a_v, sa_v, sidx2, didx2, sdidx2, rows_v, den_sh, c_sh,
             sem_g, sem_s, sem_i):
    c = lax.axis_index("c")
    s = lax.axis_index("s")
    zf = jnp.zeros((16,), jnp.float32)

    pltpu.sync_copy(td_hbm, td_v)
    pltpu.sync_copy(ts_hbm, ts_v)
    pltpu.sync_copy(lm_hbm, lm_v)
    e1 = s * EP

    def _z_rows(i, carry):
        for r in range(DH // 16):
            rows_v[0, i, pl.ds(r * 16, 16)] = zf
        return carry
    lax.fori_loop(0, SK, _z_rows, 0)

    def _z(i, carry):
        zro_v[pl.ds(i * 16, 16)] = zf
        return carry
    lax.fori_loop(0, STRIPE // 16, _z, 0)

    row0 = s * STRIPE
    pltpu.sync_copy(zro_v, den_sh.at[pl.ds(row0, STRIPE)])
    pltpu.sync_copy(rows_v.at[0], c_sh.at[pl.ds(row0, SK)])
    pltpu.sync_copy(rows_v.at[0, pl.ds(0, STRIPE - SK)],
                    c_sh.at[pl.ds(row0 + SK, STRIPE - SK)])
    plsc.subcore_barrier()

    lm = lm_v[...]

    def _fetch(j, b):
        off = e1 + j * SK
        pltpu.async_copy(ei_hbm.at[0, pl.ds(off, SK)], srcc_v.at[b], sem_i)
        pltpu.async_copy(ei_hbm.at[1, pl.ds(off, SK)], dstc_v.at[b], sem_i)

    def _prep(j, b):
        off = e1 + j * SK
        pltpu.make_async_copy(ei_hbm.at[0, pl.ds(off, SK)], srcc_v.at[b], sem_i).wait()
        pltpu.make_async_copy(ei_hbm.at[1, pl.ds(off, SK)], dstc_v.at[b], sem_i).wait()
        for g in range(GS):
            sl = pl.ds(g * 16, 16)
            d16 = dstc_v[b, sl]
            s16 = srcc_v[b, sl]
            t = plsc.load_gather(td_v, [d16]) + plsc.load_gather(ts_v, [s16])
            t = jnp.where(t >= 0.0, t, 0.01 * t)
            ex = jnp.exp(t - lm)
            a_v[b, pl.ds(g * 16, 16)] = ex
            sidx2[b, g // Q, pl.ds((g % Q) * 16, 16)] = s16
            didx2[b, g // Q, pl.ds((g % Q) * 16, 16)] = d16

    def _fire_gather(b):
        for q in range(Q):
            pltpu.async_copy(hv_hbm.at[c].at[sidx2.at[b, q]],
                             rows_v.at[b, pl.ds(q * KC, KC)], sem_g)

    def _drain_gather(b):
        for q in range(Q):
            pltpu.make_async_copy(hv_hbm.at[c].at[sidx2.at[b, q]],
                                  rows_v.at[b, pl.ds(q * KC, KC)], sem_g).wait()

    def _fire_scatter(b):
        for q in range(Q):
            for x in range(KC // 16):
                sdidx2[b, q, pl.ds(x * 16, 16)] = didx2[b, q, pl.ds(x * 16, 16)]
                sa_v[b, pl.ds(q * KC + x * 16, 16)] = a_v[b, pl.ds(q * KC + x * 16, 16)]
        for q in range(Q):
            pltpu.async_copy(rows_v.at[b, pl.ds(q * KC, KC)],
                             c_sh.at[sdidx2.at[b, q]], sem_s, add=True)
            pltpu.async_copy(sa_v.at[b, pl.ds(q * KC, KC)],
                             den_sh.at[sdidx2.at[b, q]], sem_s, add=True)

    def _drain_scatter(b):
        for q in range(Q):
            pltpu.make_async_copy(rows_v.at[b, pl.ds(q * KC, KC)],
                                  c_sh.at[sdidx2.at[b, q]], sem_s).wait()
            pltpu.make_async_copy(sa_v.at[b, pl.ds(q * KC, KC)],
                                  den_sh.at[sdidx2.at[b, q]], sem_s).wait()

    def _scale_buf(b):
        @plsc.parallel_loop(0, SK, step=1, unroll=4)
        def _scale(e):
            ae = plsc.load_gather(a_v.at[b], [jnp.zeros((16,), jnp.int32) + e])
            for r in range(DH // 16):
                sl = pl.ds(r * 16, 16)
                rows_v[b, e, sl] = rows_v[b, e, sl] * ae

    NPAIR = NSK // 2

    def _p2(jj, carry):
        j0 = jj * 2

        @pl.when(jj > 0)
        def _():
            _drain_scatter(1)
        _prep(j0 + 1, 1)
        _fire_gather(1)

        @pl.when(jj < NPAIR - 1)
        def _():
            _fetch(j0 + 2, 0)
        _drain_gather(0)
        _scale_buf(0)
        _fire_scatter(0)

        @pl.when(jj < NPAIR - 1)
        def _():
            _prep(j0 + 2, 0)
        _drain_gather(1)
        _scale_buf(1)
        _drain_scatter(0)

        @pl.when(jj < NPAIR - 1)
        def _():
            _fire_gather(0)
            _fetch(j0 + 3, 1)
        _fire_scatter(1)
        return carry

    _fetch(0, 0)
    _fetch(1, 1)
    _prep(0, 0)
    _fire_gather(0)
    lax.fori_loop(0, NPAIR, _p2, 0)
    _drain_scatter(1)

    plsc.subcore_barrier()

    @pl.when(s < NS - 1)
    def _():
        pltpu.sync_copy(c_sh.at[pl.ds(row0, STRIPE)],
                        out_hbm.at[c, pl.ds(row0, STRIPE)])
        pltpu.sync_copy(den_sh.at[pl.ds(row0, STRIPE)],
                        den_hbm.at[c, pl.ds(row0, STRIPE)])

    @pl.when(s == NS - 1)
    def _():
        pltpu.sync_copy(c_sh.at[pl.ds(row0, N - (NS - 1) * STRIPE)],
                        out_hbm.at[c, pl.ds(row0, N - (NS - 1) * STRIPE)])
        pltpu.sync_copy(den_sh.at[pl.ds(row0, N - (NS - 1) * STRIPE)],
                        den_hbm.at[c, pl.ds(row0, N - (NS - 1) * STRIPE)])


def _post_body(cp_ref, den_ref, nf_ref, w1c_ref, w1n_ref, b1_ref, w2_ref, b2_ref,
               g_ref, bt_ref, out_ref):
    den = den_ref[0]
    recip = jnp.where(den > 0.0, 1.0 / den, 0.0)
    csum = jnp.concatenate([cp_ref[0], cp_ref[1]], axis=1) * recip
    ctx = jnp.where(csum > 0.0, csum, jnp.exp(jnp.minimum(csum, 0.0)) - 1.0)
    nf = nf_ref[...]
    h = (jnp.dot(ctx, w1c_ref[...], preferred_element_type=jnp.float32)
         + jnp.dot(nf, w1n_ref[...], preferred_element_type=jnp.float32)
         + b1_ref[...])
    h = jnp.maximum(h, 0.0)
    o = jnp.dot(h, w2_ref[...], preferred_element_type=jnp.float32) + b2_ref[...]
    o = jnp.maximum(o, 0.0)
    mean = jnp.mean(o, axis=0, keepdims=True)
    var = jnp.mean((o - mean) ** 2, axis=0, keepdims=True)
    out_ref[...] = (o - mean) * (g_ref[...] * lax.rsqrt(var + 1e-5)) + bt_ref[...]


def kernel(node_feats, edge_index, W_e, b_e, W_pn, b_pn, W1, b1, W2, b2,
           gamma, beta):
    f32 = jnp.float32
    hv, td, ts, lm = pl.pallas_call(
        _pre_body,
        out_shape=[
            jax.ShapeDtypeStruct((NC, N, DH), f32),
            jax.ShapeDtypeStruct((N, 1), f32),
            jax.ShapeDtypeStruct((N, 1), f32),
            jax.ShapeDtypeStruct((1, 16), f32),
        ],
    )(node_feats, W_e, W_pn, b_pn.reshape(1, D), b_e.reshape(1, 1))

    cparts, denp = _sc_main(td.reshape(N), ts.reshape(N), lm.reshape(16), hv,
                            edge_index)

    out = pl.pallas_call(
        _post_body,
        out_shape=jax.ShapeDtypeStruct((N, D), f32),
    )(cparts, denp.reshape(NC, N, 1), node_feats, W1[:D], W1[D:],
      b1.reshape(1, D), W2, b2.reshape(1, D), gamma.reshape(1, D),
      beta.reshape(1, D))
    return out
